# baseline (speedup 1.0000x reference)
import functools

import jax
import jax.numpy as jnp
from jax import lax
from jax.experimental import pallas as pl
from jax.experimental.pallas import tpu as pltpu
from jax.experimental.pallas import tpu_sc as plsc

B = 1024
D = 16
BI = 1024


def _make_sc_gather():
    info = plsc.get_sparse_core_info()
    nc, ns = info.num_cores, info.num_subcores
    nw = nc * ns
    bpw = B // nw
    mesh = plsc.VectorSubcoreMesh(core_axis_name="c", subcore_axis_name="s")

    @functools.partial(
        pl.kernel,
        out_type=[jax.ShapeDtypeStruct((B, D), jnp.float32)] * 4,
        mesh=mesh,
        scratch_types=[
            pltpu.VMEM((bpw,), jnp.int32),
            pltpu.VMEM((bpw,), jnp.int32),
            pltpu.VMEM((bpw,), jnp.int32),
            pltpu.VMEM((bpw, D), jnp.float32),
            pltpu.VMEM((bpw, D), jnp.float32),
            pltpu.VMEM((bpw, D), jnp.float32),
            pltpu.VMEM((bpw, D), jnp.float32),
            pltpu.SemaphoreType.DMA,
        ],
    )
    def gather_kernel(ent_hbm, rel_hbm, wr_hbm, ih_hbm, ir_hbm, it_hbm,
                      h_out, r_out, t_out, w_out,
                      ih_v, ir_v, it_v, hv, rv, tv, wv, sem):
        wid = lax.axis_index("s") * nc + lax.axis_index("c")
        base = wid * bpw
        pltpu.sync_copy(ih_hbm.at[pl.ds(base, bpw)], ih_v)
        pltpu.sync_copy(ir_hbm.at[pl.ds(base, bpw)], ir_v)
        pltpu.sync_copy(it_hbm.at[pl.ds(base, bpw)], it_v)

        def row_gather(table_hbm, idx_v, dst_v):
            copies = []
            for c in range(bpw // 16):
                vec = idx_v[pl.ds(c * 16, 16)]
                for j in range(16):
                    scalar = vec[j]
                    copies.append(pltpu.async_copy(
                        table_hbm.at[pl.ds(scalar, 1), :],
                        dst_v.at[pl.ds(c * 16 + j, 1), :], sem))
            for cp in copies:
                cp.wait()

        row_gather(ent_hbm, ih_v, hv)
        row_gather(rel_hbm, ir_v, rv)
        row_gather(ent_hbm, it_v, tv)
        row_gather(wr_hbm, ir_v, wv)
        pltpu.sync_copy(hv, h_out.at[pl.ds(base, bpw)])
        pltpu.sync_copy(rv, r_out.at[pl.ds(base, bpw)])
        pltpu.sync_copy(tv, t_out.at[pl.ds(base, bpw)])
        pltpu.sync_copy(wv, w_out.at[pl.ds(base, bpw)])

    return gather_kernel


def _score_body(h_ref, r_ref, t_ref, w_ref, out_ref):
    h = h_ref[...]
    r = r_ref[:B, :]
    t = t_ref[...]
    w = w_ref[:B, :]
    g = t - h
    d = r - g
    dk = jnp.sum(g, axis=0, keepdims=True)
    w2 = w * w
    b = dk * w2
    ww = jnp.sum(w2, axis=1, keepdims=True)
    dot = lambda a, c: jax.lax.dot_general(
        a, c, (((1,), (1,)), ((), ())),
        preferred_element_type=jnp.float32)
    ones_b = jnp.ones((B, D), jnp.float32)
    v = jnp.sum(w * w2 * dk, axis=1, keepdims=True)
    m = dot(w, g)
    pq = dot(jnp.concatenate([2.0 * w, 2.0 * (w * b)], axis=1),
             jnp.concatenate([d, ones_b], axis=1))
    gnd = dot(jnp.concatenate([2.0 * b, jnp.ones((BI, D), jnp.float32),
                               b * b], axis=1),
              jnp.concatenate([d, d * d, ones_b], axis=1))
    u = m - v
    acc = (u * ww + pq) * u + gnd
    out_ref[...] = jnp.sqrt(jnp.maximum(acc, 0.0))


def _score(ht, rp, wp):
    bp = rp.shape[0]
    return pl.pallas_call(
        _score_body,
        grid=(B // BI,),
        in_specs=[pl.BlockSpec((B, D), lambda i: (0, 0)),
                  pl.BlockSpec((bp, D), lambda i: (0, 0)),
                  pl.BlockSpec((B, D), lambda i: (1, 0)),
                  pl.BlockSpec((bp, D), lambda i: (0, 0))],
        out_specs=pl.BlockSpec((BI, B), lambda i: (i, 0)),
        out_shape=jax.ShapeDtypeStruct((B, B), jnp.float32),
    )(ht, rp, ht, wp)


def kernel(pos_sample, ent_emb, rel_emb, wr_emb):
    idx_h = pos_sample[:, 0]
    idx_r = pos_sample[:, 1]
    idx_t = pos_sample[:, 2]
    idx_rp = jnp.concatenate([idx_r, idx_r[:8]])
    mode = "promise_in_bounds"
    ht = ent_emb.at[jnp.concatenate([idx_h, idx_t])].get(mode=mode)
    r = rel_emb.at[idx_rp].get(mode=mode)
    w = wr_emb.at[idx_rp].get(mode=mode)
    return _score(ht, r, w)


# speedup vs baseline: 1.0046x; 1.0046x over previous
import jax
import jax.numpy as jnp
from jax.experimental import pallas as pl

B = 1024
D = 16
BI = 1024


def _score_body(h_ref, r_ref, t_ref, w_ref, out_ref):
    h = h_ref[...]
    r = r_ref[:B, :]
    t = t_ref[...]
    w = w_ref[:B, :]
    g = t - h
    d = r - g
    dk = jnp.sum(g, axis=0, keepdims=True)
    w2 = w * w
    b = dk * w2
    ww = jnp.sum(w2, axis=1, keepdims=True)
    dot = lambda a, c: jax.lax.dot_general(
        a, c, (((1,), (1,)), ((), ())),
        preferred_element_type=jnp.float32)
    ones_b = jnp.ones((B, D), jnp.float32)
    v = jnp.sum(w * w2 * dk, axis=1, keepdims=True)
    m = dot(w, g)
    pq = dot(jnp.concatenate([2.0 * w, 2.0 * (w * b)], axis=1),
             jnp.concatenate([d, ones_b], axis=1))
    gnd = dot(jnp.concatenate([2.0 * b, jnp.ones((BI, D), jnp.float32),
                               b * b], axis=1),
              jnp.concatenate([d, d * d, ones_b], axis=1))
    u = m - v
    acc = (u * ww + pq) * u + gnd
    out_ref[...] = jnp.sqrt(jnp.maximum(acc, 0.0))


def _score(ht, rp, wp):
    bp = rp.shape[0]
    return pl.pallas_call(
        _score_body,
        grid=(B // BI,),
        in_specs=[pl.BlockSpec((B, D), lambda i: (0, 0)),
                  pl.BlockSpec((bp, D), lambda i: (0, 0)),
                  pl.BlockSpec((B, D), lambda i: (1, 0)),
                  pl.BlockSpec((bp, D), lambda i: (0, 0))],
        out_specs=pl.BlockSpec((BI, B), lambda i: (i, 0)),
        out_shape=jax.ShapeDtypeStruct((B, B), jnp.float32),
    )(ht, rp, ht, wp)


def kernel(pos_sample, ent_emb, rel_emb, wr_emb):
    idx_h = pos_sample[:, 0]
    idx_r = pos_sample[:, 1]
    idx_t = pos_sample[:, 2]
    idx_rp = jnp.concatenate([idx_r, idx_r[:16]])
    mode = "promise_in_bounds"
    ht = ent_emb.at[jnp.concatenate([idx_h, idx_t])].get(mode=mode)
    r = rel_emb.at[idx_rp].get(mode=mode)
    w = wr_emb.at[idx_rp].get(mode=mode)
    return _score(ht, r, w)
